# baseline (speedup 1.0000x reference)
import functools

import jax
import jax.numpy as jnp
from jax import lax
from jax.experimental import pallas as pl
from jax.experimental.pallas import tpu as pltpu
from jax.experimental.pallas import tpu_sc as plsc

N_NODES = 10000
D_FEAT = 256
N_EDGES = 160000
DT = 128
DP = 32
NC, NS = 2, 16
NW = NC * NS
E_PAD = 163840
CHUNK = 128
N0, N1 = 12, 68
NMAX = max(N0, N1)
IDX_PRELOAD = NMAX * CHUNK
BE = 2000


def _proj_body(x_ref, w_ref, b_ref, o_ref):
    xw = lax.dot_general(x_ref[...], w_ref[...],
                         (((1,), (0,)), ((), ())),
                         precision=lax.Precision.HIGHEST,
                         preferred_element_type=jnp.float32)
    o_ref[...] = xw + b_ref[...]


def _node_proj(x, ws, bs):
    return pl.pallas_call(
        _proj_body,
        grid=(10,),
        in_specs=[
            pl.BlockSpec((1000, D_FEAT), lambda i: (i, 0)),
            pl.BlockSpec((D_FEAT, DT), lambda i: (0, 0)),
            pl.BlockSpec((1, DT), lambda i: (0, 0)),
        ],
        out_specs=pl.BlockSpec((1000, DT), lambda i: (i, 0)),
        out_shape=jax.ShapeDtypeStruct((N_NODES, DT), jnp.float32),
    )(x, ws, bs)


def _gather_add(table, src, dst):
    mesh = plsc.VectorSubcoreMesh(core_axis_name="c", subcore_axis_name="s")

    @functools.partial(
        pl.kernel, mesh=mesh,
        out_type=jax.ShapeDtypeStruct((E_PAD * DP,), jnp.float32),
        scratch_types=[
            pltpu.VMEM((IDX_PRELOAD,), jnp.int32),
            pltpu.VMEM((IDX_PRELOAD,), jnp.int32),
            pltpu.VMEM((2, CHUNK, DT), jnp.float32),
            pltpu.VMEM((2, CHUNK, DT), jnp.float32),
            pltpu.VMEM((2, CHUNK * DP), jnp.float32),
            pltpu.SemaphoreType.DMA,
            pltpu.SemaphoreType.DMA,
            pltpu.SemaphoreType.DMA,
            pltpu.SemaphoreType.DMA,
            pltpu.SemaphoreType.DMA,
        ],
    )
    def k(table_hbm, src_hbm, dst_hbm, out_hbm,
          si_v, di_v, rows_s, rows_d, out_v, sem_i, sem_g0, sem_g1,
          sem_w0, sem_w1):
        ci = lax.axis_index("c")
        s = lax.axis_index("s")
        my_n = jnp.where(ci == 0, N0, N1)
        base_chunk = jnp.where(ci == 0, s * N0, 16 * N0 + s * N1)
        base = base_chunk * CHUNK
        CB = CHUNK * DP
        sem_g = (sem_g0, sem_g1)
        sem_w = (sem_w0, sem_w1)

        cp_si = pltpu.async_copy(src_hbm.at[pl.ds(base, IDX_PRELOAD)], si_v, sem_i)
        cp_di = pltpu.async_copy(dst_hbm.at[pl.ds(base, IDX_PRELOAD)], di_v, sem_i)
        cp_si.wait()
        cp_di.wait()

        def issue(c, b):
            isl = pl.ds(c * CHUNK, CHUNK)
            pltpu.async_copy(table_hbm.at[si_v.at[isl]], rows_s.at[b], sem_g[b])
            pltpu.async_copy(table_hbm.at[di_v.at[isl]], rows_d.at[b], sem_g[b])

        def wait_gathers(b):
            pltpu.make_async_copy(table_hbm.at[si_v.at[pl.ds(0, CHUNK)]],
                                  rows_s.at[b], sem_g[b]).wait()
            pltpu.make_async_copy(table_hbm.at[di_v.at[pl.ds(0, CHUNK)]],
                                  rows_d.at[b], sem_g[b]).wait()

        def add_and_write(c, b):
            @pl.loop(0, CHUNK, step=8)
            def _(r0):
                f0 = r0 * DP
                for u in range(8):
                    for j in (0, 16):
                        out_v.at[b, pl.ds(f0 + u * DP + j, 16)][...] = (
                            rows_s.at[b, pl.ds(r0 + u, 1), pl.ds(j, 16)][...]
                            + rows_d.at[b, pl.ds(r0 + u, 1), pl.ds(32 + j, 16)][...])[0]

            @pl.when(c >= 2)
            def _():
                pltpu.make_async_copy(
                    out_v.at[b], out_hbm.at[pl.ds(base * DP, CB)], sem_w[b]).wait()
            pltpu.async_copy(out_v.at[b],
                             out_hbm.at[pl.ds((base + c * CHUNK) * DP, CB)], sem_w[b])

        issue(0, 0)

        @pl.loop(0, my_n, step=2)
        def _(c):
            issue(c + 1, 1)
            wait_gathers(0)
            add_and_write(c, 0)

            @pl.when(c + 2 < my_n)
            def _():
                issue(c + 2, 0)
            wait_gathers(1)
            add_and_write(c + 1, 1)

        pltpu.make_async_copy(out_v.at[0], out_hbm.at[pl.ds(base * DP, CB)],
                              sem_w[0]).wait()
        pltpu.make_async_copy(out_v.at[1], out_hbm.at[pl.ds(base * DP, CB)],
                              sem_w[1]).wait()

    return k(table, src, dst)


def _leaky(v):
    return jnp.where(v >= 0, v, 0.1 * v)


def _mlp_body(e_ref, w8_ref, b8_ref, w81_ref, b81_ref, w82_ref, b82_ref,
              w9_ref, b9_ref, g_ref, o_ref):
    dn = (((1,), (0,)), ((), ()))
    h = _leaky(e_ref[...])
    h = _leaky(lax.dot_general(h, w8_ref[...], dn,
                               preferred_element_type=jnp.float32) + b8_ref[...])
    h = _leaky(lax.dot_general(h, w81_ref[...], dn,
                               preferred_element_type=jnp.float32) + b81_ref[...])
    h = _leaky(lax.dot_general(h, w82_ref[...], dn,
                               preferred_element_type=jnp.float32) + b82_ref[...])
    z = lax.dot_general(h, w9_ref[...], dn,
                        preferred_element_type=jnp.float32) + b9_ref[...]
    ez = jnp.exp(z)
    s = lax.dot_general(ez, g_ref[...], dn, preferred_element_type=jnp.float32)
    o_ref[...] = ez / s


def _mlp(e0, w8bd, b8t, w81bd, b81t, w82bd, b82t, w9bd, b9t, g16):
    full = lambda shape: pl.BlockSpec(shape, lambda i: tuple(0 for _ in shape))
    return pl.pallas_call(
        _mlp_body,
        grid=(N_EDGES // 4 // BE,),
        in_specs=[
            pl.BlockSpec((BE, DT), lambda i: (i, 0)),
            full((DT, 64)), full((1, 64)),
            full((64, 64)), full((1, 64)),
            full((64, 32)), full((1, 32)),
            full((32, 16)), full((1, 16)),
            full((16, 16)),
        ],
        out_specs=pl.BlockSpec((BE, 16), lambda i: (i, 0)),
        out_shape=jax.ShapeDtypeStruct((N_EDGES // 4, 16), jnp.float32),
    )(e0, w8bd, b8t, w81bd, b81t, w82bd, b82t, w9bd, b9t, g16)


def kernel(x, edge_index, W7, b7, W8, b8, W81, b81, W82, b82, W9, b9):
    W7a = W7[:D_FEAT]
    W7b = W7[D_FEAT:]
    ws = jnp.zeros((D_FEAT, DT), jnp.float32)
    ws = ws.at[:, :20].set(W7b)
    ws = ws.at[:, 32:52].set(W7a - W7b)
    bs = jnp.zeros((1, DT), jnp.float32).at[0, 32:52].set(b7)

    table = _node_proj(x, ws, bs)

    pad = ((0, E_PAD + IDX_PRELOAD - N_EDGES),)
    src = jnp.pad(edge_index[0], pad)
    dst = jnp.pad(edge_index[1], pad)

    e0 = _gather_add(table, src, dst)
    e0 = e0.reshape(E_PAD // 4, DT)

    w8bd = jnp.zeros((DT, 64), jnp.float32)
    w81bd = jnp.zeros((64, 64), jnp.float32)
    w82bd = jnp.zeros((64, 32), jnp.float32)
    w9bd = jnp.zeros((32, 16), jnp.float32)
    g16 = jnp.zeros((16, 16), jnp.float32)
    for g in range(4):
        w8bd = w8bd.at[g * 32:g * 32 + 20, g * 16:g * 16 + 10].set(W8)
        w81bd = w81bd.at[g * 16:g * 16 + 10, g * 16:g * 16 + 10].set(W81)
        w82bd = w82bd.at[g * 16:g * 16 + 10, g * 8:g * 8 + 5].set(W82)
        w9bd = w9bd.at[g * 8:g * 8 + 5, g * 4:g * 4 + 4].set(W9)
        g16 = g16.at[g * 4:g * 4 + 4, g * 4:g * 4 + 4].set(jnp.ones((4, 4)))
    b8t = jnp.tile(jnp.zeros((16,), jnp.float32).at[:10].set(b8), 4)[None, :]
    b81t = jnp.tile(jnp.zeros((16,), jnp.float32).at[:10].set(b81), 4)[None, :]
    b82t = jnp.tile(jnp.zeros((8,), jnp.float32).at[:5].set(b82), 4)[None, :]
    b9t = jnp.tile(b9, 4)[None, :]

    out = _mlp(e0, w8bd, b8t, w81bd, b81t, w82bd, b82t, w9bd, b9t, g16)
    return out.reshape(N_EDGES, 4)


# speedup vs baseline: 1.3144x; 1.3144x over previous
import functools

import jax
import jax.numpy as jnp
from jax import lax
from jax.experimental import pallas as pl
from jax.experimental.pallas import tpu as pltpu
from jax.experimental.pallas import tpu_sc as plsc

N_NODES = 10000
D_FEAT = 256
N_EDGES = 160000
DT = 128
DP = 32
NC, NS = 2, 16
NW = NC * NS
E_PAD = 163840
CHUNK = 128
N0, N1 = 40, 40
NMAX = max(N0, N1)
IDX_PRELOAD = NMAX * CHUNK
BE = 2000


def _proj_body(x_ref, w_ref, b_ref, o_ref):
    xw = lax.dot_general(x_ref[...], w_ref[...],
                         (((1,), (0,)), ((), ())),
                         precision=lax.Precision.HIGHEST,
                         preferred_element_type=jnp.float32)
    o_ref[...] = xw + b_ref[...]


def _node_proj(x, ws, bs):
    return pl.pallas_call(
        _proj_body,
        grid=(10,),
        in_specs=[
            pl.BlockSpec((1000, D_FEAT), lambda i: (i, 0)),
            pl.BlockSpec((D_FEAT, DT), lambda i: (0, 0)),
            pl.BlockSpec((1, DT), lambda i: (0, 0)),
        ],
        out_specs=pl.BlockSpec((1000, DT), lambda i: (i, 0)),
        out_shape=jax.ShapeDtypeStruct((N_NODES, DT), jnp.float32),
    )(x, ws, bs)


def _gather_add(table, src, dst):
    mesh = plsc.VectorSubcoreMesh(core_axis_name="c", subcore_axis_name="s")

    @functools.partial(
        pl.kernel, mesh=mesh,
        out_type=jax.ShapeDtypeStruct((E_PAD * DP,), jnp.float32),
        scratch_types=[
            pltpu.VMEM((IDX_PRELOAD,), jnp.int32),
            pltpu.VMEM((IDX_PRELOAD,), jnp.int32),
            pltpu.VMEM((2, CHUNK, DT), jnp.float32),
            pltpu.VMEM((2, CHUNK, DT), jnp.float32),
            pltpu.VMEM((2, CHUNK * DP), jnp.float32),
            pltpu.SemaphoreType.DMA,
            pltpu.SemaphoreType.DMA,
            pltpu.SemaphoreType.DMA,
            pltpu.SemaphoreType.DMA,
            pltpu.SemaphoreType.DMA,
        ],
    )
    def k(table_hbm, src_hbm, dst_hbm, out_hbm,
          si_v, di_v, rows_s, rows_d, out_v, sem_i, sem_g0, sem_g1,
          sem_w0, sem_w1):
        ci = lax.axis_index("c")
        s = lax.axis_index("s")
        my_n = jnp.where(ci == 0, N0, N1)
        base_chunk = jnp.where(ci == 0, s * N0, 16 * N0 + s * N1)
        base = base_chunk * CHUNK
        CB = CHUNK * DP
        sem_g = (sem_g0, sem_g1)
        sem_w = (sem_w0, sem_w1)

        cp_si = pltpu.async_copy(src_hbm.at[pl.ds(base, IDX_PRELOAD)], si_v, sem_i)
        cp_di = pltpu.async_copy(dst_hbm.at[pl.ds(base, IDX_PRELOAD)], di_v, sem_i)
        cp_si.wait()
        cp_di.wait()

        def issue(c, b):
            isl = pl.ds(c * CHUNK, CHUNK)
            pltpu.async_copy(table_hbm.at[si_v.at[isl]], rows_s.at[b], sem_g[b])
            pltpu.async_copy(table_hbm.at[di_v.at[isl]], rows_d.at[b], sem_g[b])

        def wait_gathers(b):
            pltpu.make_async_copy(table_hbm.at[si_v.at[pl.ds(0, CHUNK)]],
                                  rows_s.at[b], sem_g[b]).wait()
            pltpu.make_async_copy(table_hbm.at[di_v.at[pl.ds(0, CHUNK)]],
                                  rows_d.at[b], sem_g[b]).wait()

        def add_and_write(c, b):
            @pl.loop(0, CHUNK, step=8)
            def _(r0):
                f0 = r0 * DP
                for u in range(8):
                    for j in (0, 16):
                        out_v.at[b, pl.ds(f0 + u * DP + j, 16)][...] = (
                            rows_s.at[b, pl.ds(r0 + u, 1), pl.ds(j, 16)][...]
                            + rows_d.at[b, pl.ds(r0 + u, 1), pl.ds(32 + j, 16)][...])[0]

            @pl.when(c >= 2)
            def _():
                pltpu.make_async_copy(
                    out_v.at[b], out_hbm.at[pl.ds(base * DP, CB)], sem_w[b]).wait()
            pltpu.async_copy(out_v.at[b],
                             out_hbm.at[pl.ds((base + c * CHUNK) * DP, CB)], sem_w[b])

        issue(0, 0)

        @pl.loop(0, my_n, step=2)
        def _(c):
            issue(c + 1, 1)
            wait_gathers(0)
            add_and_write(c, 0)

            @pl.when(c + 2 < my_n)
            def _():
                issue(c + 2, 0)
            wait_gathers(1)
            add_and_write(c + 1, 1)

        pltpu.make_async_copy(out_v.at[0], out_hbm.at[pl.ds(base * DP, CB)],
                              sem_w[0]).wait()
        pltpu.make_async_copy(out_v.at[1], out_hbm.at[pl.ds(base * DP, CB)],
                              sem_w[1]).wait()

    return k(table, src, dst)


def _leaky(v):
    return jnp.where(v >= 0, v, 0.1 * v)


def _mlp_body(e_ref, w8_ref, b8_ref, w81_ref, b81_ref, w82_ref, b82_ref,
              w9_ref, b9_ref, g_ref, o_ref):
    dn = (((1,), (0,)), ((), ()))
    h = _leaky(e_ref[...])
    h = _leaky(lax.dot_general(h, w8_ref[...], dn,
                               preferred_element_type=jnp.float32) + b8_ref[...])
    h = _leaky(lax.dot_general(h, w81_ref[...], dn,
                               preferred_element_type=jnp.float32) + b81_ref[...])
    h = _leaky(lax.dot_general(h, w82_ref[...], dn,
                               preferred_element_type=jnp.float32) + b82_ref[...])
    z = lax.dot_general(h, w9_ref[...], dn,
                        preferred_element_type=jnp.float32) + b9_ref[...]
    ez = jnp.exp(z)
    s = lax.dot_general(ez, g_ref[...], dn, preferred_element_type=jnp.float32)
    o_ref[...] = ez / s


def _mlp(e0, w8bd, b8t, w81bd, b81t, w82bd, b82t, w9bd, b9t, g16):
    full = lambda shape: pl.BlockSpec(shape, lambda i: tuple(0 for _ in shape))
    return pl.pallas_call(
        _mlp_body,
        grid=(N_EDGES // 4 // BE,),
        in_specs=[
            pl.BlockSpec((BE, DT), lambda i: (i, 0)),
            full((DT, 64)), full((1, 64)),
            full((64, 64)), full((1, 64)),
            full((64, 32)), full((1, 32)),
            full((32, 16)), full((1, 16)),
            full((16, 16)),
        ],
        out_specs=pl.BlockSpec((BE, 16), lambda i: (i, 0)),
        out_shape=jax.ShapeDtypeStruct((N_EDGES // 4, 16), jnp.float32),
    )(e0, w8bd, b8t, w81bd, b81t, w82bd, b82t, w9bd, b9t, g16)


def kernel(x, edge_index, W7, b7, W8, b8, W81, b81, W82, b82, W9, b9):
    W7a = W7[:D_FEAT]
    W7b = W7[D_FEAT:]
    ws = jnp.zeros((D_FEAT, DT), jnp.float32)
    ws = ws.at[:, :20].set(W7b)
    ws = ws.at[:, 32:52].set(W7a - W7b)
    bs = jnp.zeros((1, DT), jnp.float32).at[0, 32:52].set(b7)

    table = _node_proj(x, ws, bs)

    pad = ((0, E_PAD + IDX_PRELOAD - N_EDGES),)
    src = jnp.pad(edge_index[0], pad)
    dst = jnp.pad(edge_index[1], pad)

    e0 = _gather_add(table, src, dst)
    e0 = e0.reshape(E_PAD // 4, DT)

    w8bd = jnp.zeros((DT, 64), jnp.float32)
    w81bd = jnp.zeros((64, 64), jnp.float32)
    w82bd = jnp.zeros((64, 32), jnp.float32)
    w9bd = jnp.zeros((32, 16), jnp.float32)
    g16 = jnp.zeros((16, 16), jnp.float32)
    for g in range(4):
        w8bd = w8bd.at[g * 32:g * 32 + 20, g * 16:g * 16 + 10].set(W8)
        w81bd = w81bd.at[g * 16:g * 16 + 10, g * 16:g * 16 + 10].set(W81)
        w82bd = w82bd.at[g * 16:g * 16 + 10, g * 8:g * 8 + 5].set(W82)
        w9bd = w9bd.at[g * 8:g * 8 + 5, g * 4:g * 4 + 4].set(W9)
        g16 = g16.at[g * 4:g * 4 + 4, g * 4:g * 4 + 4].set(jnp.ones((4, 4)))
    b8t = jnp.tile(jnp.zeros((16,), jnp.float32).at[:10].set(b8), 4)[None, :]
    b81t = jnp.tile(jnp.zeros((16,), jnp.float32).at[:10].set(b81), 4)[None, :]
    b82t = jnp.tile(jnp.zeros((8,), jnp.float32).at[:5].set(b82), 4)[None, :]
    b9t = jnp.tile(b9, 4)[None, :]

    out = _mlp(e0, w8bd, b8t, w81bd, b81t, w82bd, b82t, w9bd, b9t, g16)
    return out.reshape(N_EDGES, 4)
